# baseline (speedup 1.0000x reference)
import functools

import jax
import jax.numpy as jnp
from jax import lax
from jax.experimental import pallas as pl
from jax.experimental.pallas import tpu as pltpu
from jax.experimental.pallas import tpu_sc as plsc

B = 16384
F = 16
NC = 2
NS = 16
NW = NC * NS
BPW = B // NW
GROUPS = BPW // 16


def _mf_body(ue_h, ie_h, ub_h, ib_h, uids_h, iids_h, out_h,
             uidx_v, iidx_v, uf_v, if_v, ub_v, ib_v, out_v,
             s0, s1, s2, s3):
    wid = lax.axis_index("s") * NC + lax.axis_index("c")
    base = wid * BPW

    pltpu.sync_copy(uids_h.at[pl.ds(base, BPW)], uidx_v)
    pltpu.sync_copy(iids_h.at[pl.ds(base, BPW)], iidx_v)

    ucps = [pltpu.async_copy(ue_h.at[f].at[uidx_v], uf_v.at[f], s0)
            for f in range(F)]
    icps = [pltpu.async_copy(ie_h.at[f].at[iidx_v], if_v.at[f], s1)
            for f in range(F)]
    cub = pltpu.async_copy(ub_h.at[uidx_v], ub_v, s2)
    cib = pltpu.async_copy(ib_h.at[iidx_v], ib_v, s3)
    for c in ucps:
        c.wait()
    for c in icps:
        c.wait()
    cub.wait()
    cib.wait()

    def group(g, carry):
        sl = pl.ds(g * 16, 16)
        acc = ub_v[sl] + ib_v[sl]
        for f in range(F):
            acc = acc + uf_v[f, sl] * if_v[f, sl]
        out_v[sl] = acc
        return carry

    lax.fori_loop(0, GROUPS, group, 0)
    pltpu.sync_copy(out_v, out_h.at[pl.ds(base, BPW)])


@jax.jit
def _mf(uids, iids, user_embeddings, item_embeddings, user_bias, item_bias):
    mesh = plsc.VectorSubcoreMesh(core_axis_name="c", subcore_axis_name="s",
                                  num_cores=NC, num_subcores=NS)
    return pl.kernel(
        _mf_body,
        out_type=jax.ShapeDtypeStruct((B,), jnp.float32),
        mesh=mesh,
        compiler_params=pltpu.CompilerParams(
            needs_layout_passes=False, use_tc_tiling_on_sc=False),
        scratch_types=[
            pltpu.VMEM((BPW,), jnp.int32),
            pltpu.VMEM((BPW,), jnp.int32),
            pltpu.VMEM((F, BPW), jnp.float32),
            pltpu.VMEM((F, BPW), jnp.float32),
            pltpu.VMEM((BPW,), jnp.float32),
            pltpu.VMEM((BPW,), jnp.float32),
            pltpu.VMEM((BPW,), jnp.float32),
            pltpu.SemaphoreType.DMA,
            pltpu.SemaphoreType.DMA,
            pltpu.SemaphoreType.DMA,
            pltpu.SemaphoreType.DMA,
        ],
    )(user_embeddings.T, item_embeddings.T,
      user_bias.reshape(-1), item_bias.reshape(-1), uids, iids)


def kernel(uids, iids, user_embeddings, item_embeddings, user_bias, item_bias):
    return _mf(uids, iids, user_embeddings, item_embeddings,
               user_bias, item_bias)


# speedup vs baseline: 3.2596x; 3.2596x over previous
import functools

import jax
import jax.numpy as jnp
from jax import lax
from jax.experimental import pallas as pl
from jax.experimental.pallas import tpu as pltpu
from jax.experimental.pallas import tpu_sc as plsc

B = 16384
F = 16
NC = 2
NS = 16
NW = NC * NS
BPW = B // NW
WAVE = 256
ROWS_PER_BLOCK = 8


def _mf_body(ue_h, ie_h, ub_h, ib_h, uids_h, iids_h, out_h,
             uidx_v, iidx_v, ublk_v, iblk_v, ur_v, ir_v, ub_v, ib_v, o_v,
             s0, s1, s2, s3):
    wid = lax.axis_index("s") * NC + lax.axis_index("c")
    base = wid * BPW

    pltpu.sync_copy(uids_h.at[pl.ds(base, BPW)], uidx_v)
    pltpu.sync_copy(iids_h.at[pl.ds(base, BPW)], iidx_v)
    cub = pltpu.async_copy(ub_h.at[uidx_v], ub_v, s2)
    cib = pltpu.async_copy(ib_h.at[iidx_v], ib_v, s3)

    lanes = lax.iota(jnp.int32, 16)

    for w in range(BPW // WAVE):
        def mkblk(idx_ref, blk_ref, w=w):
            def body(g, carry):
                sl_src = pl.ds(w * WAVE + g * 16, 16)
                blk_ref[pl.ds(g * 16, 16)] = lax.shift_right_logical(
                    idx_ref[sl_src], 3)
                return carry
            lax.fori_loop(0, WAVE // 16, body, 0)

        mkblk(uidx_v, ublk_v)
        cu = pltpu.async_copy(ue_h.at[ublk_v], ur_v, s0)
        mkblk(iidx_v, iblk_v)
        ci = pltpu.async_copy(ie_h.at[iblk_v], ir_v, s1)
        cu.wait()
        ci.wait()

        def group(g, carry, w=w):
            sl_src = pl.ds(w * WAVE + g * 16, 16)
            uid = uidx_v[sl_src]
            iid = iidx_v[sl_src]
            rows = g * 16 + lanes
            ucol0 = (uid & 7) * 16
            icol0 = (iid & 7) * 16
            acc = jnp.zeros((16,), jnp.float32)
            for f in range(F):
                uval = plsc.load_gather(ur_v, [rows, ucol0 + f])
                ival = plsc.load_gather(ir_v, [rows, icol0 + f])
                acc = acc + uval * ival
            o_v[sl_src] = acc
            return carry
        lax.fori_loop(0, WAVE // 16, group, 0)

    cub.wait()
    cib.wait()

    def addb(g, carry):
        sl = pl.ds(g * 16, 16)
        o_v[sl] = o_v[sl] + ub_v[sl] + ib_v[sl]
        return carry
    lax.fori_loop(0, BPW // 16, addb, 0)
    pltpu.sync_copy(o_v, out_h.at[pl.ds(base, BPW)])


@jax.jit
def _mf(uids, iids, user_embeddings, item_embeddings, user_bias, item_bias):
    mesh = plsc.VectorSubcoreMesh(core_axis_name="c", subcore_axis_name="s",
                                  num_cores=NC, num_subcores=NS)
    n_users = user_embeddings.shape[0]
    n_items = item_embeddings.shape[0]
    return pl.kernel(
        _mf_body,
        out_type=jax.ShapeDtypeStruct((B,), jnp.float32),
        mesh=mesh,
        compiler_params=pltpu.CompilerParams(
            needs_layout_passes=False, use_tc_tiling_on_sc=True),
        scratch_types=[
            pltpu.VMEM((BPW,), jnp.int32),
            pltpu.VMEM((BPW,), jnp.int32),
            pltpu.VMEM((WAVE,), jnp.int32),
            pltpu.VMEM((WAVE,), jnp.int32),
            pltpu.VMEM((WAVE, 128), jnp.float32),
            pltpu.VMEM((WAVE, 128), jnp.float32),
            pltpu.VMEM((BPW,), jnp.float32),
            pltpu.VMEM((BPW,), jnp.float32),
            pltpu.VMEM((BPW,), jnp.float32),
            pltpu.SemaphoreType.DMA,
            pltpu.SemaphoreType.DMA,
            pltpu.SemaphoreType.DMA,
            pltpu.SemaphoreType.DMA,
        ],
    )(user_embeddings.reshape(n_users * F // 128, 128),
      item_embeddings.reshape(n_items * F // 128, 128),
      user_bias.reshape(-1), item_bias.reshape(-1), uids, iids)


def kernel(uids, iids, user_embeddings, item_embeddings, user_bias, item_bias):
    return _mf(uids, iids, user_embeddings, item_embeddings,
               user_bias, item_bias)


# speedup vs baseline: 11.9315x; 3.6605x over previous
import functools

import jax
import jax.numpy as jnp
from jax import lax
from jax.experimental import pallas as pl
from jax.experimental.pallas import tpu as pltpu
from jax.experimental.pallas import tpu_sc as plsc

B = 16384
F = 16
NC = 2
NS = 16
NW = NC * NS
BPW = B // NW
WV = 16


def _mf_body(ue_h, ie_h, ub_h, ib_h, uids_h, iids_h, out_h,
             uidx_v, iidx_v, ublk_v, iblk_v, ub_v, ib_v, o_v,
             s0, s1, s2, s3):
    wid = lax.axis_index("s") * NC + lax.axis_index("c")
    base = wid * BPW

    pltpu.sync_copy(uids_h.at[pl.ds(base, BPW)], uidx_v)
    pltpu.sync_copy(iids_h.at[pl.ds(base, BPW)], iidx_v)
    cub = pltpu.async_copy(ub_h.at[uidx_v], ub_v, s2)
    cib = pltpu.async_copy(ib_h.at[iidx_v], ib_v, s3)

    lanes = lax.iota(jnp.int32, 16)

    def wave(w, carry):
        sl = pl.ds(w * WV, WV)
        uids16 = uidx_v[sl]
        iids16 = iidx_v[sl]
        ut16 = lax.shift_right_logical(uids16, 7) * 128
        it16 = lax.shift_right_logical(iids16, 7) * 128
        for j in range(WV):
            uc = pl.multiple_of(ut16[j], 128)
            ic = pl.multiple_of(it16[j], 128)
            pltpu.async_copy(ue_h.at[:, pl.ds(uc, 128)], ublk_v.at[j], s0)
            pltpu.async_copy(ie_h.at[:, pl.ds(ic, 128)], iblk_v.at[j], s1)
        pltpu.make_async_copy(ue_h.at[:, pl.ds(0, WV * 128)],
                              ublk_v, s0).wait()
        pltpu.make_async_copy(ie_h.at[:, pl.ds(0, WV * 128)],
                              iblk_v, s1).wait()
        ucol = uids16 & 127
        icol = iids16 & 127
        acc = jnp.zeros((16,), jnp.float32)
        for f in range(F):
            fvec = jnp.full((16,), f, jnp.int32)
            uval = plsc.load_gather(ublk_v, [lanes, fvec, ucol])
            ival = plsc.load_gather(iblk_v, [lanes, fvec, icol])
            acc = acc + uval * ival
        o_v[sl] = acc
        return carry

    lax.fori_loop(0, BPW // WV, wave, 0)
    cub.wait()
    cib.wait()

    def addb(g, carry):
        sl = pl.ds(g * 16, 16)
        o_v[sl] = o_v[sl] + ub_v[sl] + ib_v[sl]
        return carry

    lax.fori_loop(0, BPW // 16, addb, 0)
    pltpu.sync_copy(o_v, out_h.at[pl.ds(base, BPW)])


@jax.jit
def _mf(uids, iids, user_embeddings, item_embeddings, user_bias, item_bias):
    mesh = plsc.VectorSubcoreMesh(core_axis_name="c", subcore_axis_name="s",
                                  num_cores=NC, num_subcores=NS)
    return pl.kernel(
        _mf_body,
        out_type=jax.ShapeDtypeStruct((B,), jnp.float32),
        mesh=mesh,
        compiler_params=pltpu.CompilerParams(
            needs_layout_passes=False, use_tc_tiling_on_sc=True),
        scratch_types=[
            pltpu.VMEM((BPW,), jnp.int32),
            pltpu.VMEM((BPW,), jnp.int32),
            pltpu.VMEM((WV, F, 128), jnp.float32),
            pltpu.VMEM((WV, F, 128), jnp.float32),
            pltpu.VMEM((BPW,), jnp.float32),
            pltpu.VMEM((BPW,), jnp.float32),
            pltpu.VMEM((BPW,), jnp.float32),
            pltpu.SemaphoreType.DMA,
            pltpu.SemaphoreType.DMA,
            pltpu.SemaphoreType.DMA,
            pltpu.SemaphoreType.DMA,
        ],
    )(user_embeddings.T, item_embeddings.T,
      user_bias.reshape(-1), item_bias.reshape(-1), uids, iids)


def kernel(uids, iids, user_embeddings, item_embeddings, user_bias, item_bias):
    return _mf(uids, iids, user_embeddings, item_embeddings,
               user_bias, item_bias)
